# Initial kernel scaffold; baseline (speedup 1.0000x reference)
import jax
import jax.numpy as jnp
from jax.experimental import pallas as pl


def kernel(outputs, labels):
    raise NotImplementedError("write your pallas kernel here")


# speedup vs baseline: 1.0318x; 1.0318x over previous
import functools

import jax
import jax.numpy as jnp
from jax.experimental import pallas as pl
from jax.experimental.pallas import tpu as pltpu

_LANE = 128


def _round_up(x, m):
    return (x + m - 1) // m * m


def _dice_partial_kernel(o_ref, l_ref, acc_ref, *, tr, half, nc):
    @pl.when(pl.program_id(1) == 0)
    def _init():
        acc_ref[...] = jnp.zeros_like(acc_ref)

    o = o_ref[...].astype(jnp.float32)
    l = l_ref[...].astype(jnp.float32)

    num = jnp.sum(o * l, axis=1, keepdims=True)
    den = jnp.sum(o * o + l, axis=1, keepdims=True)
    ratio = (num + num) / den

    row0 = (pl.program_id(0) * half + pl.program_id(1)) * tr
    rows = row0 + jax.lax.broadcasted_iota(jnp.int32, (tr, 1), 0)
    ratio = jnp.where(rows < nc, ratio, 0.0)

    acc_ref[...] += jnp.sum(ratio)


def kernel(outputs, labels):
    n, c, h, w = outputs.shape
    nc, hw = n * c, h * w

    o2 = outputs.reshape(nc, hw)
    l2 = labels.reshape(nc, hw)

    tr = 16 if nc % 32 == 0 else 8
    nc_pad = _round_up(nc, 2 * tr)
    hw_pad = _round_up(hw, _LANE)
    if nc_pad != nc or hw_pad != hw:
        o2 = jnp.pad(o2, ((0, nc_pad - nc), (0, hw_pad - hw)))
        l2 = jnp.pad(l2, ((0, nc_pad - nc), (0, hw_pad - hw)))

    half = nc_pad // tr // 2
    body = functools.partial(_dice_partial_kernel, tr=tr, half=half, nc=nc)

    acc = pl.pallas_call(
        body,
        out_shape=jax.ShapeDtypeStruct((2, 1, _LANE), jnp.float32),
        grid_spec=pltpu.PrefetchScalarGridSpec(
            num_scalar_prefetch=0,
            grid=(2, half),
            in_specs=[
                pl.BlockSpec((tr, hw_pad), lambda i, j: (i * half + j, 0)),
                pl.BlockSpec((tr, hw_pad), lambda i, j: (i * half + j, 0)),
            ],
            out_specs=pl.BlockSpec((1, 1, _LANE), lambda i, j: (i, 0, 0)),
        ),
        compiler_params=pltpu.CompilerParams(
            dimension_semantics=("parallel", "arbitrary"),
            vmem_limit_bytes=48 * 1024 * 1024,
        ),
    )(o2, l2)

    total = acc[0, 0, 0] + acc[1, 0, 0]
    return (1.0 - 0.5 * total / nc).astype(jnp.float32)


# speedup vs baseline: 3.6553x; 3.5426x over previous
import functools

import jax
import jax.numpy as jnp
from jax.experimental import pallas as pl
from jax.experimental.pallas import tpu as pltpu

_LANE = 128


def _dice_partial_kernel(o_ref, l_ref, acc_ref, *, c):
    @pl.when(pl.program_id(1) == 0)
    def _init():
        acc_ref[...] = jnp.zeros_like(acc_ref)

    acc = jnp.float32(0.0)
    for ci in range(c):
        o = o_ref[0, ci].astype(jnp.float32)
        l = l_ref[0, ci].astype(jnp.float32)
        num = jnp.sum(o * l)
        den = jnp.sum(o * o + l)
        acc += (num + num) / den
    acc_ref[...] += acc


def kernel(outputs, labels):
    n, c, h, w = outputs.shape
    half = n // 2

    body = functools.partial(_dice_partial_kernel, c=c)

    acc = pl.pallas_call(
        body,
        out_shape=jax.ShapeDtypeStruct((2, 1, _LANE), jnp.float32),
        grid_spec=pltpu.PrefetchScalarGridSpec(
            num_scalar_prefetch=0,
            grid=(2, half),
            in_specs=[
                pl.BlockSpec((1, c, h, w), lambda i, j: (i * half + j, 0, 0, 0)),
                pl.BlockSpec((1, c, h, w), lambda i, j: (i * half + j, 0, 0, 0)),
            ],
            out_specs=pl.BlockSpec((1, 1, _LANE), lambda i, j: (i, 0, 0)),
        ),
        compiler_params=pltpu.CompilerParams(
            dimension_semantics=("parallel", "arbitrary"),
            vmem_limit_bytes=48 * 1024 * 1024,
        ),
    )(outputs, labels)

    total = acc[0, 0, 0] + acc[1, 0, 0]
    return (1.0 - 0.5 * total / (n * c)).astype(jnp.float32)


# speedup vs baseline: 4.1570x; 1.1373x over previous
import functools

import jax
import jax.numpy as jnp
from jax.experimental import pallas as pl
from jax.experimental.pallas import tpu as pltpu

_LANE = 128


def _dice_partial_kernel(o_ref, l_ref, acc_ref, *, nb, c):
    @pl.when(pl.program_id(1) == 0)
    def _init():
        acc_ref[...] = jnp.zeros_like(acc_ref)

    acc = jnp.float32(0.0)
    for ni in range(nb):
        for ci in range(c):
            o = o_ref[ni, ci].astype(jnp.float32)
            l = l_ref[ni, ci].astype(jnp.float32)
            num = jnp.sum(o * l)
            den = jnp.sum(o * o + l)
            acc += (num + num) / den
    acc_ref[...] += acc


def kernel(outputs, labels):
    n, c, h, w = outputs.shape
    nb = 2 if n % 4 == 0 else 1
    half = n // nb // 2

    body = functools.partial(_dice_partial_kernel, nb=nb, c=c)

    acc = pl.pallas_call(
        body,
        out_shape=jax.ShapeDtypeStruct((2, 1, _LANE), jnp.float32),
        grid_spec=pltpu.PrefetchScalarGridSpec(
            num_scalar_prefetch=0,
            grid=(2, half),
            in_specs=[
                pl.BlockSpec((nb, c, h, w), lambda i, j: (i * half + j, 0, 0, 0)),
                pl.BlockSpec((nb, c, h, w), lambda i, j: (i * half + j, 0, 0, 0)),
            ],
            out_specs=pl.BlockSpec((1, 1, _LANE), lambda i, j: (i, 0, 0)),
        ),
        compiler_params=pltpu.CompilerParams(
            dimension_semantics=("parallel", "arbitrary"),
            vmem_limit_bytes=48 * 1024 * 1024,
        ),
    )(outputs, labels)

    total = acc[0, 0, 0] + acc[1, 0, 0]
    return (1.0 - 0.5 * total / (n * c)).astype(jnp.float32)
